# Initial kernel scaffold; baseline (speedup 1.0000x reference)
import jax
import jax.numpy as jnp
from jax.experimental import pallas as pl


def kernel(input_ids, attention_mask, vocab_table):
    raise NotImplementedError("write your pallas kernel here")


# speedup vs baseline: 7.7558x; 7.7558x over previous
import functools

import jax
import jax.numpy as jnp
from jax import lax
from jax.experimental import pallas as pl
from jax.experimental.pallas import tpu as pltpu
from jax.experimental.pallas import tpu_sc as plsc

VOCAB = 100000
EMBED_DIM = 128
BATCH = 1024
SEQ = 200

NC = 2
NS = 16
NW = NC * NS

TOTAL = BATCH * SEQ
PER_W = TOTAL // NW
CHUNK = 128
NCH = PER_W // CHUNK
HALF = NCH // 2

_mesh = plsc.VectorSubcoreMesh(core_axis_name="c", subcore_axis_name="s")


@functools.partial(
    pl.kernel,
    out_type=jax.ShapeDtypeStruct((TOTAL, EMBED_DIM), jnp.float32),
    mesh=_mesh,
    scratch_types=[
        pltpu.VMEM((NCH, CHUNK), jnp.int32),
        pltpu.VMEM((CHUNK, EMBED_DIM), jnp.float32),
        pltpu.VMEM((CHUNK, EMBED_DIM), jnp.float32),
        pltpu.SemaphoreType.DMA,
        pltpu.SemaphoreType.DMA,
        pltpu.SemaphoreType.DMA,
        pltpu.SemaphoreType.DMA,
    ],
)
def _gather_kernel(table_hbm, ids_hbm, out_hbm,
                   idx_v, buf0, buf1, g0, g1, s0, s1):
    wid = lax.axis_index("s") * NC + lax.axis_index("c")
    obase = wid * PER_W

    pltpu.sync_copy(ids_hbm.at[wid], idx_v)

    def start_gather(ch, buf, sem):
        pltpu.async_copy(table_hbm.at[idx_v.at[ch]], buf, sem)

    def wait_gather(buf, sem):
        pltpu.make_async_copy(table_hbm.at[pl.ds(0, CHUNK)], buf, sem).wait()

    def start_scatter(ch, buf, sem):
        pltpu.async_copy(buf, out_hbm.at[pl.ds(obase + ch * CHUNK, CHUNK)], sem)

    def wait_scatter(buf, sem):
        pltpu.make_async_copy(buf, out_hbm.at[pl.ds(obase, CHUNK)], sem).wait()

    start_gather(0, buf0, g0)

    def body(t, carry):
        a = 2 * t

        @pl.when(t > 0)
        def _w1():
            wait_scatter(buf1, s1)

        start_gather(a + 1, buf1, g1)
        wait_gather(buf0, g0)
        start_scatter(a, buf0, s0)
        wait_scatter(buf0, s0)

        @pl.when(t < HALF - 1)
        def _w2():
            start_gather(a + 2, buf0, g0)

        wait_gather(buf1, g1)
        start_scatter(a + 1, buf1, s1)
        return carry

    lax.fori_loop(0, HALF, body, 0)
    wait_scatter(buf1, s1)


def _mask_body(am_ref, out_ref):
    out_ref[...] = am_ref[...] != 1


def kernel(input_ids, attention_mask, vocab_table):
    ids = input_ids.astype(jnp.int32).reshape(NW, NCH, CHUNK)
    gathered = _gather_kernel(vocab_table, ids)
    mask = pl.pallas_call(
        _mask_body,
        out_shape=jax.ShapeDtypeStruct((BATCH, SEQ), jnp.bool_),
    )(attention_mask)
    return gathered.reshape(BATCH, SEQ, EMBED_DIM), mask
